# baseline (speedup 1.0000x reference)
import functools

import jax
import jax.numpy as jnp
from jax import lax
from jax.experimental import pallas as pl
from jax.experimental.pallas import tpu as pltpu
from jax.experimental.pallas import tpu_sc as plsc

VOCAB = 1000
B, S = 1024, 50
N = B * S
NC, NS = 2, 16
NW = NC * NS
R = N // NW
L = 16
V = R // L
CH = 80
NCH = R // CH


def _body(txt_hbm, tgt_hbm, vis_hbm, out_hbm, tgtbuf, idx_v, gath_v, vis_v,
          part_v, gsem, vsem):
    wid = lax.axis_index("s") * NC + lax.axis_index("c")
    base = wid * R

    vis_cp = pltpu.make_async_copy(vis_hbm.at[pl.ds(base, R)], vis_v, vsem)
    vis_cp.start()
    pltpu.sync_copy(tgt_hbm.at[pl.ds(base, R)], tgtbuf.at[pl.ds(8, R)])

    @pl.loop(0, V)
    def _compute_idx(j):
        cur = tgtbuf[pl.ds(8 + j * L, L)]
        tc = jnp.where(cur > VOCAB, 0, cur)
        rows = (base + j * L) + lax.iota(jnp.int32, L)
        idx_v[pl.ds(j * L, L)] = rows * (VOCAB + 1) + tc

    gather_cps = []
    for c in range(NCH):
        cp = pltpu.make_async_copy(
            txt_hbm.at[idx_v.at[pl.ds(c * CH, CH)]],
            gath_v.at[pl.ds(c * CH, CH)], gsem)
        cp.start()
        gather_cps.append(cp)
    for cp in gather_cps:
        cp.wait()
    vis_cp.wait()

    zero = jnp.zeros((L,), jnp.float32)

    def acc_step(j, carry):
        a_txt, a_vis, a_cnt = carry
        cur = tgtbuf[pl.ds(8 + j * L, L)]
        prev = tgtbuf[pl.ds(7 + j * L, L)]
        pos = j * L + lax.iota(jnp.int32, L)
        vis_f = jnp.where(cur > VOCAB, 1.0, 0.0)
        first_f = jnp.where(pos % S == 0, 1.0, 0.0)
        prev_f = jnp.where(prev > 0, 1.0, 0.0)
        txt_f = (1.0 - vis_f) * jnp.minimum(first_f + prev_f, 1.0)
        g = gath_v[pl.ds(j * L, L)]
        vv = vis_v[pl.ds(j * L, L)]
        a_txt = a_txt + g * txt_f
        a_vis = a_vis + vv * vis_f
        a_cnt = a_cnt + txt_f + vis_f
        return a_txt, a_vis, a_cnt

    a_txt, a_vis, a_cnt = lax.fori_loop(0, V, acc_step, (zero, zero, zero))

    part_v[pl.ds(0, L)] = a_txt
    part_v[pl.ds(L, L)] = a_vis
    part_v[pl.ds(2 * L, L)] = a_cnt
    pltpu.sync_copy(part_v, out_hbm.at[pl.ds(wid * 3 * L, 3 * L)])


@jax.jit
def kernel(txt_input, vis_input, target):
    txt_flat = txt_input.reshape(-1)
    vis_flat = vis_input.reshape(-1)
    tgt_flat = target.reshape(-1)

    mesh = plsc.VectorSubcoreMesh(
        core_axis_name="c", subcore_axis_name="s",
        num_cores=NC, num_subcores=NS)
    run = pl.kernel(
        _body,
        out_type=jax.ShapeDtypeStruct((NW * 3 * L,), jnp.float32),
        mesh=mesh,
        scratch_types=[
            pltpu.VMEM((R + 8,), jnp.int32),
            pltpu.VMEM((R,), jnp.int32),
            pltpu.VMEM((R,), jnp.float32),
            pltpu.VMEM((R,), jnp.float32),
            pltpu.VMEM((3 * L,), jnp.float32),
            pltpu.SemaphoreType.DMA,
            pltpu.SemaphoreType.DMA,
        ],
    )
    parts = run(txt_flat, tgt_flat, vis_flat).reshape(NW, 3, L)
    sums = jnp.sum(parts, axis=(0, 2))
    return -(sums[0] + sums[1]) / sums[2]
